# Initial kernel scaffold; baseline (speedup 1.0000x reference)
import functools

import jax
import jax.numpy as jnp
from jax import lax
from jax.experimental import pallas as pl
from jax.experimental.pallas import tpu as pltpu
from jax.experimental.pallas import tpu_sc as plsc

VOCAB = 1000000
DIM = 64
BATCH = 16384
HIST = 50

NC = 2
NS = 16
NW = NC * NS

B = BATCH * HIST
B_PER_W = B // NW
G = 128
GROUPS_PER_CHUNK = 4
C = G * GROUPS_PER_CHUNK
CHUNKS = B_PER_W // C


@functools.partial(
    pl.kernel,
    out_type=jax.ShapeDtypeStruct((B, DIM), jnp.float32),
    mesh=plsc.VectorSubcoreMesh(core_axis_name="c", subcore_axis_name="s"),
    scratch_types=[
        pltpu.VMEM((GROUPS_PER_CHUNK, G), jnp.int32),
        pltpu.VMEM((C, DIM), jnp.float32),
        pltpu.SemaphoreType.DMA,
    ],
)
def _gather_kernel(idx_hbm, table_hbm, out_hbm, idx_v, rows_v, sem):
    wid = lax.axis_index("s") * NC + lax.axis_index("c")
    base_g = wid * (B_PER_W // G)

    def chunk(i, _):
        g0 = base_g + i * GROUPS_PER_CHUNK
        pltpu.sync_copy(idx_hbm.at[pl.ds(g0, GROUPS_PER_CHUNK)], idx_v)
        for j in range(GROUPS_PER_CHUNK):
            pltpu.async_copy(
                table_hbm.at[idx_v.at[j]],
                rows_v.at[pl.ds(j * G, G)],
                sem,
            ).wait()
        pltpu.sync_copy(rows_v, out_hbm.at[pl.ds(g0 * G, C)])
        return 0

    lax.fori_loop(0, CHUNKS, chunk, 0)


def kernel(x, weight):
    idx = x.reshape(B // G, G).astype(jnp.int32)
    out = _gather_kernel(idx, weight)
    return out.reshape(BATCH, HIST, DIM)


# speedup vs baseline: 1.6715x; 1.6715x over previous
import functools

import jax
import jax.numpy as jnp
from jax import lax
from jax.experimental import pallas as pl
from jax.experimental.pallas import tpu as pltpu
from jax.experimental.pallas import tpu_sc as plsc

VOCAB = 1000000
DIM = 64
BATCH = 16384
HIST = 50

NC = 2
NS = 16
NW = NC * NS

B = BATCH * HIST
B_PER_W = B // NW
G = 128
GROUPS_PER_CHUNK = 4
C = G * GROUPS_PER_CHUNK
CHUNKS = B_PER_W // C


@functools.partial(
    pl.kernel,
    out_type=jax.ShapeDtypeStruct((B, DIM), jnp.float32),
    mesh=plsc.VectorSubcoreMesh(core_axis_name="c", subcore_axis_name="s"),
    scratch_types=[
        pltpu.VMEM((GROUPS_PER_CHUNK, G), jnp.int32),
        pltpu.VMEM((C, DIM), jnp.float32),
        pltpu.SemaphoreType.DMA,
    ],
    compiler_params=pltpu.CompilerParams(use_tc_tiling_on_sc=False),
)
def _gather_kernel(idx_hbm, table_hbm, out_hbm, idx_v, rows_v, sem):
    wid = lax.axis_index("s") * NC + lax.axis_index("c")
    base_g = wid * (B_PER_W // G)

    def chunk(i, _):
        g0 = base_g + i * GROUPS_PER_CHUNK
        pltpu.sync_copy(idx_hbm.at[pl.ds(g0, GROUPS_PER_CHUNK)], idx_v)
        for j in range(GROUPS_PER_CHUNK):
            pltpu.async_copy(
                table_hbm.at[idx_v.at[j]],
                rows_v.at[pl.ds(j * G, G)],
                sem,
            ).wait()
        pltpu.sync_copy(rows_v, out_hbm.at[pl.ds(g0 * G, C)])
        return 0

    lax.fori_loop(0, CHUNKS, chunk, 0)


def kernel(x, weight):
    idx = x.reshape(B // G, G).astype(jnp.int32)
    out = _gather_kernel(idx, weight)
    return out.reshape(BATCH, HIST, DIM)


# speedup vs baseline: 1.8688x; 1.1180x over previous
import functools

import jax
import jax.numpy as jnp
from jax import lax
from jax.experimental import pallas as pl
from jax.experimental.pallas import tpu as pltpu
from jax.experimental.pallas import tpu_sc as plsc

VOCAB = 1000000
DIM = 64
BATCH = 16384
HIST = 50

NC = 2
NS = 16
NW = NC * NS

B = BATCH * HIST
B_PER_W = B // NW
G = 128
GPC = 2
C = G * GPC
CHUNKS = B_PER_W // C
GROUPS = B_PER_W // G
K = 4
T = CHUNKS // K


@functools.partial(
    pl.kernel,
    out_type=jax.ShapeDtypeStruct((B, DIM), jnp.float32),
    mesh=plsc.VectorSubcoreMesh(core_axis_name="c", subcore_axis_name="s"),
    scratch_types=[
        pltpu.VMEM((GROUPS, G), jnp.int32),
        [pltpu.VMEM((C, DIM), jnp.float32) for _ in range(K)],
        [pltpu.SemaphoreType.DMA for _ in range(K)],
        [pltpu.SemaphoreType.DMA for _ in range(K)],
    ],
    compiler_params=pltpu.CompilerParams(use_tc_tiling_on_sc=False),
)
def _gather_kernel(idx_hbm, table_hbm, out_hbm, idx_v, rows, sem_g, sem_o):
    wid = lax.axis_index("s") * NC + lax.axis_index("c")
    base_g = wid * GROUPS
    base_r = base_g * G

    def fire_gathers(b, k):
        return [
            pltpu.async_copy(
                table_hbm.at[idx_v.at[k * GPC + j]],
                rows[b].at[pl.ds(j * G, G)],
                sem_g[b],
            )
            for j in range(GPC)
        ]

    def fire_out(b, k):
        pltpu.async_copy(rows[b], out_hbm.at[pl.ds(base_r + k * C, C)], sem_o[b])

    def drain_out(b, k):
        pltpu.make_async_copy(
            rows[b], out_hbm.at[pl.ds(base_r + k * C, C)], sem_o[b]
        ).wait()

    pltpu.sync_copy(idx_hbm.at[pl.ds(base_g, GROUPS)], idx_v)

    ds = [fire_gathers(b, b) for b in range(K)]
    for b in range(K):
        for d in ds[b]:
            d.wait()
        fire_out(b, b)

    def body(t, _):
        k0 = t * K
        ds = []
        for b in range(K):
            drain_out(b, k0 - K + b)
            ds.append(fire_gathers(b, k0 + b))
        for b in range(K):
            for d in ds[b]:
                d.wait()
            fire_out(b, k0 + b)
        return 0

    lax.fori_loop(1, T, body, 0)

    for b in range(K):
        drain_out(b, (T - 1) * K + b)


def kernel(x, weight):
    idx = x.reshape(B // G, G).astype(jnp.int32)
    out = _gather_kernel(idx, weight)
    return out.reshape(BATCH, HIST, DIM)
